# baseline (speedup 1.0000x reference)
import functools

import jax
import jax.numpy as jnp
from jax import lax
from jax.experimental import pallas as pl
from jax.experimental.pallas import tpu as pltpu
from jax.experimental.pallas import tpu_sc as plsc

B = 16384
F = 26
EMBD = 32
N = B * F

NC = 2
NS = 16
NW = NC * NS
CHUNK = 128
N_ROWS = N // CHUNK
ROWS_PER_W = N_ROWS // NW
SUP = 4
N_SUP = ROWS_PER_W // SUP
LANES = 16


def kernel(xi, xv, ca_emb_weight):
    xi_t = xi.T.reshape(N_ROWS, CHUNK).astype(jnp.int32)
    xv_t = xv.T.reshape(N_ROWS, CHUNK)

    mesh = plsc.VectorSubcoreMesh(core_axis_name="c", subcore_axis_name="s")

    @functools.partial(
        pl.kernel,
        out_type=jax.ShapeDtypeStruct((F, EMBD // 8, B // CHUNK, 8, CHUNK),
                                      jnp.float32),
        mesh=mesh,
        scratch_types=[
            pltpu.VMEM((ROWS_PER_W, CHUNK), jnp.int32),
            pltpu.VMEM((ROWS_PER_W, CHUNK), jnp.float32),
            pltpu.VMEM((SUP, CHUNK, EMBD), jnp.float32),
            pltpu.VMEM((SUP, CHUNK, EMBD), jnp.float32),
            pltpu.VMEM((EMBD // 8, SUP, 8, CHUNK), jnp.float32),
            pltpu.VMEM((EMBD // 8, SUP, 8, CHUNK), jnp.float32),
            pltpu.SemaphoreType.DMA((2,)),
            pltpu.SemaphoreType.DMA((2,)),
        ],
        compiler_params=pltpu.CompilerParams(
            use_tc_tiling_on_sc=False, needs_layout_passes=False
        ),
    )
    def k(table_hbm, idx_hbm, xv_hbm, out_hbm,
          idx_v, xv_v, gbuf0, gbuf1, obuf0, obuf1, gsem, wsem):
        gbuf = (gbuf0, gbuf1)
        obuf = (obuf0, obuf1)
        wid = lax.axis_index("s") * NC + lax.axis_index("c")
        rbase = wid * ROWS_PER_W

        pltpu.sync_copy(idx_hbm.at[pl.ds(rbase, ROWS_PER_W)], idx_v)
        pltpu.sync_copy(xv_hbm.at[pl.ds(rbase, ROWS_PER_W)], xv_v)

        iota16 = lax.iota(jnp.int32, LANES)
        jsplat = [jnp.full((LANES,), j, jnp.int32) for j in range(SUP)]
        esplat = [jnp.full((LANES,), e, jnp.int32) for e in range(EMBD)]

        def start_gathers(i, b):
            for j in range(SUP):
                pltpu.async_copy(
                    table_hbm.at[idx_v.at[i * SUP + j]],
                    gbuf[b].at[j],
                    gsem.at[b],
                )

        def wait_gathers(i, b):
            for j in range(SUP):
                pltpu.make_async_copy(
                    table_hbm.at[idx_v.at[i * SUP + j]],
                    gbuf[b].at[j],
                    gsem.at[b],
                ).wait()

        def out_slice(i, tr):
            r0 = rbase + i * SUP
            f = lax.shift_right_logical(r0, 7)
            bb0 = lax.bitwise_and(r0, 127)
            return out_hbm.at[f, tr, pl.ds(bb0, SUP)]

        def start_writebacks(i, b):
            for tr in range(EMBD // 8):
                pltpu.async_copy(obuf[b].at[tr], out_slice(i, tr), wsem.at[b])

        def wait_writebacks(i, b):
            for tr in range(EMBD // 8):
                pltpu.make_async_copy(
                    obuf[b].at[tr], out_slice(i, tr), wsem.at[b]
                ).wait()

        def compute(i, b):
            g_ref, o_ref = gbuf[b], obuf[b]

            @pl.loop(0, CHUNK // LANES)
            def _(bl0):
                lane0 = bl0 * LANES
                row_idx = lane0 + iota16
                for j in range(SUP):
                    xvv = xv_v[i * SUP + j, pl.ds(lane0, LANES)]
                    for e in range(EMBD):
                        g = plsc.load_gather(
                            g_ref, [jsplat[j], row_idx, esplat[e]]
                        )
                        o_ref.at[e // 8, j, e % 8, pl.ds(lane0, LANES)][...] = (
                            g * xvv
                        )

        start_gathers(0, 0)

        @pl.loop(0, N_SUP, step=2)
        def _(i0):
            for b in range(2):
                i = i0 + b
                wait_gathers(i, b)
                @pl.when(i + 1 < N_SUP)
                def _():
                    start_gathers(i + 1, 1 - b)
                @pl.when(i >= 2)
                def _():
                    wait_writebacks(i - 2, b)
                compute(i, b)
                start_writebacks(i, b)

        for b in range(2):
            wait_writebacks(N_SUP - 2 + b, b)

    out5d = k(ca_emb_weight, xi_t, xv_t)
    return jnp.transpose(out5d, (2, 4, 0, 1, 3)).reshape(B, F, EMBD)


# speedup vs baseline: 1.0737x; 1.0737x over previous
import functools

import jax
import jax.numpy as jnp
from jax import lax
from jax.experimental import pallas as pl
from jax.experimental.pallas import tpu as pltpu
from jax.experimental.pallas import tpu_sc as plsc

B = 16384
F = 26
EMBD = 32
N = B * F

NC = 2
NS = 16
NW = NC * NS
CHUNK = 128
N_ROWS = N // CHUNK
ROWS_PER_W = N_ROWS // NW
SUP = 4
N_SUP = ROWS_PER_W // SUP
LANES = 16


def kernel(xi, xv, ca_emb_weight):
    xi_t = xi.T.reshape(N_ROWS, CHUNK).astype(jnp.int32)
    xv_t = xv.T.reshape(N_ROWS, CHUNK)

    mesh = plsc.VectorSubcoreMesh(core_axis_name="c", subcore_axis_name="s")

    @functools.partial(
        pl.kernel,
        out_type=jax.ShapeDtypeStruct((F, EMBD // 8, B // CHUNK, 8, CHUNK),
                                      jnp.float32),
        mesh=mesh,
        scratch_types=[
            pltpu.VMEM((ROWS_PER_W, CHUNK), jnp.int32),
            pltpu.VMEM((ROWS_PER_W, CHUNK), jnp.float32),
            pltpu.VMEM((SUP, CHUNK, EMBD), jnp.float32),
            pltpu.VMEM((SUP, CHUNK, EMBD), jnp.float32),
            pltpu.VMEM((EMBD // 8, SUP, 8, CHUNK), jnp.float32),
            pltpu.VMEM((EMBD // 8, SUP, 8, CHUNK), jnp.float32),
            pltpu.SemaphoreType.DMA((2,)),
            pltpu.SemaphoreType.DMA((2,)),
        ],
        compiler_params=pltpu.CompilerParams(
            use_tc_tiling_on_sc=False, needs_layout_passes=False
        ),
    )
    def k(table_hbm, idx_hbm, xv_hbm, out_hbm,
          idx_v, xv_v, gbuf0, gbuf1, obuf0, obuf1, gsem, wsem):
        gbuf = (gbuf0, gbuf1)
        obuf = (obuf0, obuf1)
        wid = lax.axis_index("s") * NC + lax.axis_index("c")
        rbase = wid * ROWS_PER_W

        pltpu.sync_copy(idx_hbm.at[pl.ds(rbase, ROWS_PER_W)], idx_v)
        pltpu.sync_copy(xv_hbm.at[pl.ds(rbase, ROWS_PER_W)], xv_v)

        iota16 = lax.iota(jnp.int32, LANES)
        jsplat = [jnp.full((LANES,), j, jnp.int32) for j in range(SUP)]
        esplat = [jnp.full((LANES,), e, jnp.int32) for e in range(EMBD)]

        def start_gathers(i, b):
            for j in range(SUP):
                pltpu.async_copy(
                    table_hbm.at[idx_v.at[i * SUP + j]],
                    gbuf[b].at[j],
                    gsem.at[b],
                )

        def wait_gathers(i, b):
            for j in range(SUP):
                pltpu.make_async_copy(
                    table_hbm.at[idx_v.at[i * SUP + j]],
                    gbuf[b].at[j],
                    gsem.at[b],
                ).wait()

        def out_slice(i, tr):
            r0 = rbase + i * SUP
            f = lax.shift_right_logical(r0, 7)
            bb0 = lax.bitwise_and(r0, 127)
            return out_hbm.at[f, tr, pl.ds(bb0, SUP)]

        def start_writebacks(i, b):
            for tr in range(EMBD // 8):
                pltpu.async_copy(obuf[b].at[tr], out_slice(i, tr), wsem.at[b])

        def wait_writebacks(i, b):
            for tr in range(EMBD // 8):
                pltpu.make_async_copy(
                    obuf[b].at[tr], out_slice(i, tr), wsem.at[b]
                ).wait()

        def compute(i, b):
            g_ref, o_ref = gbuf[b], obuf[b]

            @plsc.parallel_loop(0, CHUNK // LANES, unroll=2)
            def _(bl0):
                lane0 = bl0 * LANES
                row_idx = lane0 + iota16
                for j in range(SUP):
                    xvv = xv_v[i * SUP + j, pl.ds(lane0, LANES)]
                    for e in range(EMBD):
                        g = plsc.load_gather(
                            g_ref, [jsplat[j], row_idx, esplat[e]]
                        )
                        o_ref.at[e // 8, j, e % 8, pl.ds(lane0, LANES)][...] = (
                            g * xvv
                        )

        start_gathers(0, 0)

        @pl.loop(0, N_SUP, step=2)
        def _(i0):
            for b in range(2):
                i = i0 + b
                wait_gathers(i, b)
                @pl.when(i + 1 < N_SUP)
                def _():
                    start_gathers(i + 1, 1 - b)
                @pl.when(i >= 2)
                def _():
                    wait_writebacks(i - 2, b)
                compute(i, b)
                start_writebacks(i, b)

        for b in range(2):
            wait_writebacks(N_SUP - 2 + b, b)

    out5d = k(ca_emb_weight, xi_t, xv_t)
    return jnp.transpose(out5d, (2, 4, 0, 1, 3)).reshape(B, F, EMBD)
